# baseline (speedup 1.0000x reference)
import jax
import jax.numpy as jnp
from jax.experimental import pallas as pl


def _copy_body(inputs_ref, buffer_ref, out_ref):
    i = pl.program_id(0)
    bo, seg, d = out_ref.shape

    @pl.when(i == 0)
    def _():
        out_ref[...] = inputs_ref[...].reshape(bo, seg, d)

    @pl.when(i > 0)
    def _():
        out_ref[...] = buffer_ref[...].reshape(bo, seg, d)


def kernel(inputs, buffer):
    b, d = inputs.shape
    n_steps = buffer.shape[0]
    seg = n_steps // b
    sub = 8
    rows_in = b // sub
    rows_all = n_steps // sub
    n_blocks = rows_all // rows_in
    bo = b // seg

    inputs3 = inputs.reshape(rows_in, sub, d)
    buffer3 = buffer.reshape(rows_all, sub, d)

    return pl.pallas_call(
        _copy_body,
        grid=(n_blocks,),
        in_specs=[
            pl.BlockSpec((rows_in, sub, d), lambda i: (0, 0, 0)),
            pl.BlockSpec((rows_in, sub, d), lambda i: (jnp.maximum(i - 1, 0), 0, 0)),
        ],
        out_specs=pl.BlockSpec((bo, seg, d), lambda i: (i, 0, 0)),
        out_shape=jax.ShapeDtypeStruct((b, seg, d), inputs.dtype),
    )(inputs3, buffer3)


# speedup vs baseline: 1.0021x; 1.0021x over previous
import jax
import jax.numpy as jnp
from jax.experimental import pallas as pl


def _copy_body(inputs_ref, buffer_ref, out_ref):
    i = pl.program_id(0)

    @pl.when(i == 0)
    def _():
        out_ref[...] = inputs_ref[...]

    @pl.when(i > 0)
    def _():
        out_ref[...] = buffer_ref[...]


def kernel(inputs, buffer):
    b, d = inputs.shape
    n_steps = buffer.shape[0]
    seg = n_steps // b
    in_outer = b // seg
    n_outer = n_steps // seg
    n_blocks = n_outer // in_outer

    inputs3 = inputs.reshape(in_outer, seg, d)
    buffer3 = buffer.reshape(n_outer, seg, d)

    return pl.pallas_call(
        _copy_body,
        grid=(n_blocks,),
        in_specs=[
            pl.BlockSpec((in_outer, seg, d), lambda i: (0, 0, 0)),
            pl.BlockSpec((in_outer, seg, d), lambda i: (jnp.maximum(i - 1, 0), 0, 0)),
        ],
        out_specs=pl.BlockSpec((in_outer, seg, d), lambda i: (i, 0, 0)),
        out_shape=jax.ShapeDtypeStruct((b, seg, d), inputs.dtype),
    )(inputs3, buffer3)
